# baseline (speedup 1.0000x reference)
import functools

import jax
import jax.numpy as jnp
from jax import lax
from jax.experimental import pallas as pl
from jax.experimental.pallas import tpu as pltpu
from jax.experimental.pallas import tpu_sc as plsc

HIDDEN = 4096
NUM_CORES = 2
NUM_SUBCORES = 16
NUM_WORKERS = NUM_CORES * NUM_SUBCORES
CHUNK = 8


def _build(num_rows, batch):
    rows_per_w = num_rows // NUM_WORKERS
    nchunk = rows_per_w // CHUNK
    s_per_w = rows_per_w // batch
    seq = num_rows // batch
    mesh = plsc.VectorSubcoreMesh(core_axis_name="c", subcore_axis_name="s")

    @functools.partial(
        pl.kernel,
        mesh=mesh,
        out_type=jax.ShapeDtypeStruct((num_rows // 4, 4, HIDDEN), jnp.float32),
        scratch_types=[
            pltpu.VMEM((rows_per_w,), jnp.int32),
            pltpu.VMEM((rows_per_w,), jnp.int32),
            pltpu.VMEM((3, CHUNK, HIDDEN), jnp.float32),
            pltpu.SemaphoreType.DMA,
            pltpu.SemaphoreType.DMA,
        ],
    )
    def gather_kernel(ids_hbm, table_hbm, out3_hbm, idx_bs, idx_v, bufs, gsem, wsem):
        out_hbm = out3_hbm.reshape(num_rows, HIDDEN)
        wid = lax.axis_index("s") * NUM_CORES + lax.axis_index("c")
        base = wid * rows_per_w
        s0 = wid * s_per_w

        for bb in range(batch):
            pltpu.sync_copy(
                ids_hbm.at[pl.ds(bb * seq + s0, s_per_w)],
                idx_bs.at[pl.ds(bb * s_per_w, s_per_w)],
            )
        lane = lax.iota(jnp.int32, 16)
        sub = lane >> 2
        bsel = lane & (batch - 1)
        for v4 in range(s_per_w // 16):
            xs = [
                idx_bs[pl.ds(b * s_per_w + 16 * v4, 16)] for b in range(batch)
            ]
            for t in range(4):
                pos = 4 * t + sub
                dnums = lax.GatherDimensionNumbers(
                    offset_dims=(), collapsed_slice_dims=(0,),
                    start_index_map=(0,),
                )
                ys = [
                    lax.gather(
                        x, pos[:, None], dnums, (1,),
                        mode=lax.GatherScatterMode.PROMISE_IN_BOUNDS,
                    )
                    for x in xs
                ]
                tgt = ys[0]
                for b in range(1, batch):
                    tgt = jnp.where(bsel == b, ys[b], tgt)
                idx_v[pl.ds(64 * v4 + 16 * t, 16)] = tgt

        def start_gather(c, b):
            pltpu.async_copy(
                table_hbm.at[idx_v.at[pl.ds(c * CHUNK, CHUNK)]], bufs.at[b], gsem
            )

        def wait_gather(c, b):
            pltpu.make_async_copy(
                table_hbm.at[idx_v.at[pl.ds(c * CHUNK, CHUNK)]], bufs.at[b], gsem
            ).wait()

        def start_write(c, b):
            pltpu.async_copy(
                bufs.at[b], out_hbm.at[pl.ds(base + c * CHUNK, CHUNK)], wsem
            )

        def wait_write(c, b):
            pltpu.make_async_copy(
                bufs.at[b], out_hbm.at[pl.ds(base + c * CHUNK, CHUNK)], wsem
            ).wait()

        start_gather(0, 0)
        start_gather(1, 1)

        wait_gather(0, 0)
        start_write(0, 0)
        start_gather(2, 2)

        def body(i, carry):
            c0 = 1 + i * 3
            for b in range(3):
                c = c0 + b
                slot = (1 + b) % 3
                wait_gather(c, slot)
                start_write(c, slot)
                wait_write(c - 1, b)
                start_gather(c + 2, b)
            return carry

        lax.fori_loop(0, (nchunk - 5) // 3, body, 0)

        for c in (nchunk - 4, nchunk - 3):
            wait_gather(c, c % 3)
            start_write(c, c % 3)
            wait_write(c - 1, (c - 1) % 3)
            start_gather(c + 2, (c + 2) % 3)
        for c in (nchunk - 2, nchunk - 1):
            wait_gather(c, c % 3)
            start_write(c, c % 3)
            wait_write(c - 1, (c - 1) % 3)
        wait_write(nchunk - 1, (nchunk - 1) % 3)

    return gather_kernel


def kernel(input_ids, word_embeddings):
    b, s = input_ids.shape
    ids_flat = input_ids.astype(jnp.int32).reshape(-1)
    return _build(b * s, b)(ids_flat, word_embeddings)


# speedup vs baseline: 1.0128x; 1.0128x over previous
import functools

import jax
import jax.numpy as jnp
from jax import lax
from jax.experimental import pallas as pl
from jax.experimental.pallas import tpu as pltpu
from jax.experimental.pallas import tpu_sc as plsc

HIDDEN = 4096
NUM_CORES = 2
NUM_SUBCORES = 16
NUM_WORKERS = NUM_CORES * NUM_SUBCORES
CHUNK = 8


def _build(num_rows, batch):
    rows_per_w = num_rows // NUM_WORKERS
    nchunk = rows_per_w // CHUNK
    s_per_w = rows_per_w // batch
    seq = num_rows // batch
    mesh = plsc.VectorSubcoreMesh(core_axis_name="c", subcore_axis_name="s")

    @functools.partial(
        pl.kernel,
        mesh=mesh,
        out_type=jax.ShapeDtypeStruct((num_rows // 4, 4, HIDDEN), jnp.float32),
        scratch_types=[
            pltpu.VMEM((rows_per_w,), jnp.int32),
            pltpu.VMEM((rows_per_w,), jnp.int32),
            pltpu.VMEM((3, CHUNK, HIDDEN), jnp.float32),
            pltpu.SemaphoreType.DMA,
            pltpu.SemaphoreType.DMA,
        ],
    )
    def gather_kernel(ids_hbm, table_hbm, out3_hbm, idx_bs, idx_v, bufs, gsem, wsem):
        out_hbm = out3_hbm.reshape(num_rows, HIDDEN)
        wid = lax.axis_index("s") * NUM_CORES + lax.axis_index("c")
        base = wid * rows_per_w
        s0 = wid * s_per_w

        for bb in range(batch):
            pltpu.async_copy(
                ids_hbm.at[pl.ds(bb * seq + s0, s_per_w)],
                idx_bs.at[pl.ds(bb * s_per_w, s_per_w)],
                wsem,
            )
        for bb in range(batch):
            pltpu.make_async_copy(
                ids_hbm.at[pl.ds(bb * seq + s0, s_per_w)],
                idx_bs.at[pl.ds(bb * s_per_w, s_per_w)],
                wsem,
            ).wait()
        lane = lax.iota(jnp.int32, 16)
        sub = lane >> 2
        bsel = lane & (batch - 1)
        for v4 in range(s_per_w // 16):
            xs = [
                idx_bs[pl.ds(b * s_per_w + 16 * v4, 16)] for b in range(batch)
            ]
            for t in range(4):
                pos = 4 * t + sub
                dnums = lax.GatherDimensionNumbers(
                    offset_dims=(), collapsed_slice_dims=(0,),
                    start_index_map=(0,),
                )
                ys = [
                    lax.gather(
                        x, pos[:, None], dnums, (1,),
                        mode=lax.GatherScatterMode.PROMISE_IN_BOUNDS,
                    )
                    for x in xs
                ]
                tgt = ys[0]
                for b in range(1, batch):
                    tgt = jnp.where(bsel == b, ys[b], tgt)
                idx_v[pl.ds(64 * v4 + 16 * t, 16)] = tgt
            if v4 == 0:
                pltpu.async_copy(
                    table_hbm.at[idx_v.at[pl.ds(0, CHUNK)]], bufs.at[0], gsem
                )
                pltpu.async_copy(
                    table_hbm.at[idx_v.at[pl.ds(CHUNK, CHUNK)]], bufs.at[1], gsem
                )

        def start_gather(c, b):
            pltpu.async_copy(
                table_hbm.at[idx_v.at[pl.ds(c * CHUNK, CHUNK)]], bufs.at[b], gsem
            )

        def wait_gather(c, b):
            pltpu.make_async_copy(
                table_hbm.at[idx_v.at[pl.ds(c * CHUNK, CHUNK)]], bufs.at[b], gsem
            ).wait()

        def start_write(c, b):
            pltpu.async_copy(
                bufs.at[b], out_hbm.at[pl.ds(base + c * CHUNK, CHUNK)], wsem
            )

        def wait_write(c, b):
            pltpu.make_async_copy(
                bufs.at[b], out_hbm.at[pl.ds(base + c * CHUNK, CHUNK)], wsem
            ).wait()

        wait_gather(0, 0)
        start_write(0, 0)
        start_gather(2, 2)

        def body(i, carry):
            c0 = 1 + i * 3
            for b in range(3):
                c = c0 + b
                slot = (1 + b) % 3
                wait_gather(c, slot)
                start_write(c, slot)
                wait_write(c - 1, b)
                start_gather(c + 2, b)
            return carry

        lax.fori_loop(0, (nchunk - 5) // 3, body, 0)

        for c in (nchunk - 4, nchunk - 3):
            wait_gather(c, c % 3)
            start_write(c, c % 3)
            wait_write(c - 1, (c - 1) % 3)
            start_gather(c + 2, (c + 2) % 3)
        for c in (nchunk - 2, nchunk - 1):
            wait_gather(c, c % 3)
            start_write(c, c % 3)
            wait_write(c - 1, (c - 1) % 3)
        wait_write(nchunk - 1, (nchunk - 1) % 3)

    return gather_kernel


def kernel(input_ids, word_embeddings):
    b, s = input_ids.shape
    ids_flat = input_ids.astype(jnp.int32).reshape(-1)
    return _build(b * s, b)(ids_flat, word_embeddings)
